# baseline (speedup 1.0000x reference)
import functools

import jax
import jax.numpy as jnp
from jax import lax
from jax.experimental import pallas as pl
from jax.experimental.pallas import tpu as pltpu
from jax.experimental.pallas import tpu_sc as plsc

N_NODES = 10000
N_EDGES = 320000
NUM_OPS = 120
OP_DIM = 64
D_IN = 128
D_HID = 256

CHUNK = 112
NCHUNK = 3072
E_PAD = NCHUNK * CHUNK
PAD_DST = 10104
NACC = 10112
ZROWS = NACC // 16
OPS_PER_TILE = 320
NOPS_PAD = 10240

_mesh = lambda: plsc.VectorSubcoreMesh(core_axis_name="c", subcore_axis_name="s")


@functools.partial(
    pl.kernel, mesh=_mesh(),
    out_type=jax.ShapeDtypeStruct((NOPS_PAD, 128), jnp.float32),
    scratch_types=[
        pltpu.VMEM((OPS_PER_TILE,), jnp.int32),
        pltpu.VMEM((CHUNK, 128), jnp.float32),
        pltpu.SemaphoreType.DMA,
    ],
)
def _sc_op_lookup(emb, ops, out, idxv, buf, sem):
    c = lax.axis_index("c")
    s = lax.axis_index("s")
    base = (c * 16 + s) * OPS_PER_TILE
    pltpu.sync_copy(ops.at[pl.ds(base, OPS_PER_TILE)], idxv)
    for off, sz in ((0, 128), (128, 128), (256, 64)):
        pltpu.async_copy(emb.at[idxv.at[pl.ds(off, sz)]],
                         buf.at[pl.ds(0, sz)], sem).wait()
        pltpu.sync_copy(buf.at[pl.ds(0, sz)], out.at[pl.ds(base + off, sz)])


def _edge_sweep(gtable, grp, gstart, srcc, dstc, base,
                idxs, idxd, rows, acc, gsem, ssem):
    pltpu.sync_copy(srcc.at[pl.ds(base, grp)], idxs)
    pltpu.sync_copy(dstc.at[pl.ds(base, grp)], idxd)

    def gwait(j, b):
        pltpu.make_async_copy(gtable.at[idxs.at[j]], rows.at[b], gsem).wait()

    def swait(j):
        pltpu.make_async_copy(
            rows.at[lax.rem(j, 3)], acc.at[idxd.at[j]], ssem).wait()

    gstart(0, 0)

    def body(j, carry):
        b = lax.rem(j, 3)
        nb = lax.rem(j + 1, 3)

        @pl.when(j > 1)
        def _():
            swait(j - 2)

        @pl.when(j + 1 < grp)
        def _():
            gstart(j + 1, nb)

        gwait(j, b)
        pltpu.async_copy(rows.at[b], acc.at[idxd.at[j]], ssem, add=True)
        return carry

    lax.fori_loop(0, grp, body, 0)
    swait(grp - 2)
    swait(grp - 1)


_GRP1 = 24
_GRP = 24


@functools.partial(
    pl.kernel, mesh=_mesh(),
    out_type=jax.ShapeDtypeStruct((2, NACC, D_IN), jnp.float32),
    scratch_types=[
        pltpu.VMEM((_GRP1, CHUNK), jnp.int32),
        pltpu.VMEM((_GRP1, CHUNK), jnp.int32),
        pltpu.VMEM((3, CHUNK, D_IN), jnp.float32),
        pltpu.VMEM_SHARED((NACC, D_IN), jnp.float32),
        pltpu.SemaphoreType.DMA,
        pltpu.SemaphoreType.DMA,
    ],
)
def _sc_agg_l1(x0, srcc, dstc, zacc,
               aggp, idxs, idxd, rows, acc, gsem, ssem):
    c = lax.axis_index("c")
    s = lax.axis_index("s")
    t = c * 16 + s
    nck = NCHUNK // 32
    r0 = s * ZROWS
    pltpu.sync_copy(zacc.at[pl.ds(r0, ZROWS)], acc.at[pl.ds(r0, ZROWS)])
    plsc.subcore_barrier()

    def gstart(j, b):
        pltpu.async_copy(x0.at[idxs.at[j]], rows.at[b], gsem)

    def gbody(g, carry):
        _edge_sweep(x0, _GRP1, gstart, srcc, dstc, t * nck + g * _GRP1,
                    idxs, idxd, rows, acc, gsem, ssem)
        return carry

    lax.fori_loop(0, nck // _GRP1, gbody, 0)
    plsc.subcore_barrier()

    @pl.when(c == 0)
    def _():
        pltpu.sync_copy(acc.at[pl.ds(r0, ZROWS)], aggp.at[0, pl.ds(r0, ZROWS)])

    @pl.when(c == 1)
    def _():
        pltpu.sync_copy(acc.at[pl.ds(r0, ZROWS)], aggp.at[1, pl.ds(r0, ZROWS)])


@functools.partial(
    pl.kernel, mesh=_mesh(),
    out_type=jax.ShapeDtypeStruct((2, NACC, D_IN), jnp.float32),
    scratch_types=[
        pltpu.VMEM((_GRP, CHUNK), jnp.int32),
        pltpu.VMEM((_GRP, CHUNK), jnp.int32),
        pltpu.VMEM((3, CHUNK, D_IN), jnp.float32),
        pltpu.VMEM_SHARED((NACC, D_IN), jnp.float32),
        pltpu.SemaphoreType.DMA,
        pltpu.SemaphoreType.DMA,
    ],
)
def _sc_agg_h2(xlo, xhi, srcc, dstc, zacc,
               aggs, idxs, idxd, rows, acc, gsem, ssem):
    c = lax.axis_index("c")
    s = lax.axis_index("s")
    nck = NCHUNK // 16
    r0 = s * ZROWS
    pltpu.sync_copy(zacc.at[pl.ds(r0, ZROWS)], acc.at[pl.ds(r0, ZROWS)])
    plsc.subcore_barrier()

    def gstart(j, b):
        @pl.when(c == 0)
        def _():
            pltpu.async_copy(xlo.at[idxs.at[j]], rows.at[b], gsem)

        @pl.when(c == 1)
        def _():
            pltpu.async_copy(xhi.at[idxs.at[j]], rows.at[b], gsem)

    def gbody(g, carry):
        _edge_sweep(xlo, _GRP, gstart, srcc, dstc, s * nck + g * _GRP,
                    idxs, idxd, rows, acc, gsem, ssem)
        return carry

    lax.fori_loop(0, nck // _GRP, gbody, 0)
    plsc.subcore_barrier()

    @pl.when(c == 0)
    def _():
        pltpu.sync_copy(acc.at[pl.ds(r0, ZROWS)], aggs.at[0, pl.ds(r0, ZROWS)])

    @pl.when(c == 1)
    def _():
        pltpu.sync_copy(acc.at[pl.ds(r0, ZROWS)], aggs.at[1, pl.ds(r0, ZROWS)])


_BLK = 400
_EBLK = 8192
_NQ = 80


def _tc_deg_body(dstb, o):
    i = pl.program_id(0)
    dv = dstb[0]
    q = dv // 128
    r = dv % 128
    m1 = (lax.broadcasted_iota(jnp.int32, (_NQ, _EBLK), 0) == q
          ).astype(jnp.bfloat16)
    m2 = (lax.broadcasted_iota(jnp.int32, (128, _EBLK), 0) == r
          ).astype(jnp.bfloat16)
    part = lax.dot_general(m1, m2, (((1,), (1,)), ((), ())),
                           preferred_element_type=jnp.float32)

    @pl.when(i == 0)
    def _():
        o[...] = part

    @pl.when(i > 0)
    def _():
        o[...] = o[...] + part


_tc_deg = pl.pallas_call(
    _tc_deg_body,
    grid=(E_PAD // _EBLK,),
    in_specs=[pl.BlockSpec((1, 1, _EBLK), lambda i: (i, 0, 0))],
    out_specs=pl.BlockSpec((_NQ, 128), lambda i: (0, 0)),
    out_shape=jax.ShapeDtypeStruct((_NQ, 128), jnp.float32),
)


def _dotT(a, w):
    return lax.dot_general(a, w, (((1,), (1,)), ((), ())),
                           preferred_element_type=jnp.float32)


def _norm_act(h):
    n = jnp.sqrt(jnp.sum(h * h, axis=1, keepdims=True))
    h = h / jnp.maximum(n, 1e-12)
    return jnp.where(h > 0, h, 0.01 * h)


def _tc1_body(x0, aggp, degp, wl, wr, b, olo, ohi):
    deg = degp[0, 0]
    dinv = 1.0 / jnp.maximum(deg, 1.0)
    agg = (aggp[0] + aggp[1]) * dinv[:, None]
    h = _dotT(agg, wl[...]) + _dotT(x0[...], wr[...]) + b[...]
    h = _norm_act(h)
    olo[...] = h[:, :D_IN]
    ohi[...] = h[:, D_IN:]


_tc1 = pl.pallas_call(
    _tc1_body,
    grid=(N_NODES // _BLK,),
    in_specs=[
        pl.BlockSpec((_BLK, D_IN), lambda i: (i, 0)),
        pl.BlockSpec((2, _BLK, D_IN), lambda i: (0, i, 0)),
        pl.BlockSpec((1, 1, _BLK), lambda i: (i, 0, 0)),
        pl.BlockSpec((D_HID, D_IN), lambda i: (0, 0)),
        pl.BlockSpec((D_HID, D_IN), lambda i: (0, 0)),
        pl.BlockSpec((1, D_HID), lambda i: (0, 0)),
    ],
    out_specs=[pl.BlockSpec((_BLK, D_IN), lambda i: (i, 0)),
               pl.BlockSpec((_BLK, D_IN), lambda i: (i, 0))],
    out_shape=[jax.ShapeDtypeStruct((N_NODES, D_IN), jnp.float32),
               jax.ShapeDtypeStruct((N_NODES, D_IN), jnp.float32)],
)


def _make_tc23(final):
    def body(xlo, xhi, aggs, degp, wl, wr, b, *outs):
        deg = degp[0, 0]
        dinv = 1.0 / jnp.maximum(deg, 1.0)
        agg = jnp.concatenate([aggs[0], aggs[1]], axis=1) * dinv[:, None]
        x = jnp.concatenate([xlo[...], xhi[...]], axis=1)
        h = _dotT(agg, wl[...]) + _dotT(x, wr[...]) + b[...]
        if final:
            outs[0][...] = h
        else:
            h = _norm_act(h)
            outs[0][...] = h[:, :D_IN]
            outs[1][...] = h[:, D_IN:]

    in_specs = [
        pl.BlockSpec((_BLK, D_IN), lambda i: (i, 0)),
        pl.BlockSpec((_BLK, D_IN), lambda i: (i, 0)),
        pl.BlockSpec((2, _BLK, D_IN), lambda i: (0, i, 0)),
        pl.BlockSpec((1, 1, _BLK), lambda i: (i, 0, 0)),
        pl.BlockSpec((D_HID, D_HID), lambda i: (0, 0)),
        pl.BlockSpec((D_HID, D_HID), lambda i: (0, 0)),
        pl.BlockSpec((1, D_HID), lambda i: (0, 0)),
    ]
    if final:
        out_specs = pl.BlockSpec((_BLK, D_HID), lambda i: (i, 0))
        out_shape = jax.ShapeDtypeStruct((N_NODES, D_HID), jnp.float32)
    else:
        out_specs = [pl.BlockSpec((_BLK, D_IN), lambda i: (i, 0)),
                     pl.BlockSpec((_BLK, D_IN), lambda i: (i, 0))]
        out_shape = [jax.ShapeDtypeStruct((N_NODES, D_IN), jnp.float32),
                     jax.ShapeDtypeStruct((N_NODES, D_IN), jnp.float32)]
    return pl.pallas_call(body, grid=(N_NODES // _BLK,), in_specs=in_specs,
                          out_specs=out_specs, out_shape=out_shape)


_tc2 = _make_tc23(final=False)
_tc3 = _make_tc23(final=True)


def kernel(node_features, node_separation, node_ops, edges, batches,
           opcode_emb, W1l, b1, W1r, W2l, b2, W2r, W3l, b3, W3r):
    del node_separation, batches

    src = edges[0].astype(jnp.int32)
    dst = edges[1].astype(jnp.int32)
    pad_e = E_PAD - N_EDGES

    def _chunked(v, fill):
        c = jnp.concatenate(
            [v, jnp.full((pad_e,), fill, jnp.int32)]).reshape(NCHUNK, CHUNK)
        return c.reshape(NCHUNK // 32, 32, CHUNK).transpose(1, 0, 2).reshape(
            NCHUNK, CHUNK)

    srcc = _chunked(src, 0)
    dstc = _chunked(dst, PAD_DST)
    ops_p = jnp.concatenate(
        [node_ops.astype(jnp.int32),
         jnp.zeros((NOPS_PAD - N_NODES,), jnp.int32)])
    zacc = jnp.zeros((NACC, D_IN), jnp.float32)

    emb_p = jnp.concatenate(
        [opcode_emb, jnp.zeros((NUM_OPS, 128 - OP_DIM), jnp.float32)], axis=1)
    op_e = _sc_op_lookup(emb_p, ops_p)[:N_NODES, :OP_DIM]
    x0 = jnp.concatenate([node_features, op_e], axis=1)

    degq = _tc_deg(dstc.reshape(E_PAD // _EBLK, 1, _EBLK))
    degt = degq.reshape(_NQ * 128)[:N_NODES].reshape(N_NODES // _BLK, 1, _BLK)

    aggp = _sc_agg_l1(x0, srcc, dstc, zacc)
    x1lo, x1hi = _tc1(x0, aggp, degt, W1l, W1r, b1.reshape(1, D_HID))

    aggs2 = _sc_agg_h2(x1lo, x1hi, srcc, dstc, zacc)
    x2lo, x2hi = _tc2(x1lo, x1hi, aggs2, degt, W2l, W2r, b2.reshape(1, D_HID))

    aggs3 = _sc_agg_h2(x2lo, x2hi, srcc, dstc, zacc)
    return _tc3(x2lo, x2hi, aggs3, degt, W3l, W3r, b3.reshape(1, D_HID))


# speedup vs baseline: 2.2342x; 2.2342x over previous
import functools

import jax
import jax.numpy as jnp
from jax import lax
from jax.experimental import pallas as pl
from jax.experimental.pallas import tpu as pltpu
from jax.experimental.pallas import tpu_sc as plsc

N_NODES = 10000
N_EDGES = 320000
NUM_OPS = 120
OP_DIM = 64
D_IN = 128
D_HID = 256

CHUNK = 128
NCHUNK = 2560
E_PAD = NCHUNK * CHUNK
PAD_DST = 10104
NACC = 10112
ZROWS = NACC // 16
OPS_PER_TILE = 320
NOPS_PAD = 10240

_mesh = lambda: plsc.VectorSubcoreMesh(core_axis_name="c", subcore_axis_name="s")


@functools.partial(
    pl.kernel, mesh=_mesh(),
    out_type=jax.ShapeDtypeStruct((NOPS_PAD, 128), jnp.float32),
    scratch_types=[
        pltpu.VMEM((OPS_PER_TILE,), jnp.int32),
        pltpu.VMEM((CHUNK, 128), jnp.float32),
        pltpu.SemaphoreType.DMA,
    ],
)
def _sc_op_lookup(emb, ops, out, idxv, buf, sem):
    c = lax.axis_index("c")
    s = lax.axis_index("s")
    base = (c * 16 + s) * OPS_PER_TILE
    pltpu.sync_copy(ops.at[pl.ds(base, OPS_PER_TILE)], idxv)
    for off, sz in ((0, 128), (128, 128), (256, 64)):
        pltpu.async_copy(emb.at[idxv.at[pl.ds(off, sz)]],
                         buf.at[pl.ds(0, sz)], sem).wait()
        pltpu.sync_copy(buf.at[pl.ds(0, sz)], out.at[pl.ds(base + off, sz)])


def _edge_sweep(gtable, grp, gstart, srcc, dstc, base,
                idxs, idxd, rows, acc, gsem, ssem):
    pltpu.sync_copy(srcc.at[pl.ds(base, grp)], idxs)
    pltpu.sync_copy(dstc.at[pl.ds(base, grp)], idxd)

    def gwait(j, b):
        pltpu.make_async_copy(gtable.at[idxs.at[j]], rows.at[b], gsem).wait()

    gstart(0, 0)

    def body(j, carry):
        b = lax.rem(j, 2)
        nb = lax.rem(j + 1, 2)

        @pl.when(j > 0)
        def _():
            pltpu.make_async_copy(
                rows.at[nb], acc.at[idxd.at[j - 1]], ssem).wait()

        @pl.when(j + 1 < grp)
        def _():
            gstart(j + 1, nb)

        gwait(j, b)
        pltpu.async_copy(rows.at[b], acc.at[idxd.at[j]], ssem, add=True)
        return carry

    lax.fori_loop(0, grp, body, 0)
    pltpu.make_async_copy(
        rows.at[(grp - 1) % 2], acc.at[idxd.at[grp - 1]], ssem).wait()


_GRP1 = 40
_GRP = 32


@functools.partial(
    pl.kernel, mesh=_mesh(),
    out_type=jax.ShapeDtypeStruct((2, NACC, D_IN), jnp.float32),
    scratch_types=[
        pltpu.VMEM((_GRP1, CHUNK), jnp.int32),
        pltpu.VMEM((_GRP1, CHUNK), jnp.int32),
        pltpu.VMEM((2, CHUNK, D_IN), jnp.float32),
        pltpu.VMEM_SHARED((NACC, D_IN), jnp.float32),
        pltpu.SemaphoreType.DMA,
        pltpu.SemaphoreType.DMA,
    ],
)
def _sc_agg_l1(x0, srcc, dstc, zacc,
               aggp, idxs, idxd, rows, acc, gsem, ssem):
    c = lax.axis_index("c")
    s = lax.axis_index("s")
    t = c * 16 + s
    nck = NCHUNK // 32
    r0 = s * ZROWS
    pltpu.sync_copy(zacc.at[pl.ds(r0, ZROWS)], acc.at[pl.ds(r0, ZROWS)])
    plsc.subcore_barrier()

    def gstart(j, b):
        pltpu.async_copy(x0.at[idxs.at[j]], rows.at[b], gsem)

    def gbody(g, carry):
        _edge_sweep(x0, _GRP1, gstart, srcc, dstc, t * nck + g * _GRP1,
                    idxs, idxd, rows, acc, gsem, ssem)
        return carry

    lax.fori_loop(0, nck // _GRP1, gbody, 0)
    plsc.subcore_barrier()

    @pl.when(c == 0)
    def _():
        pltpu.sync_copy(acc.at[pl.ds(r0, ZROWS)], aggp.at[0, pl.ds(r0, ZROWS)])

    @pl.when(c == 1)
    def _():
        pltpu.sync_copy(acc.at[pl.ds(r0, ZROWS)], aggp.at[1, pl.ds(r0, ZROWS)])


@functools.partial(
    pl.kernel, mesh=_mesh(),
    out_type=jax.ShapeDtypeStruct((2, NACC, D_IN), jnp.float32),
    scratch_types=[
        pltpu.VMEM((_GRP, CHUNK), jnp.int32),
        pltpu.VMEM((_GRP, CHUNK), jnp.int32),
        pltpu.VMEM((2, CHUNK, D_IN), jnp.float32),
        pltpu.VMEM_SHARED((NACC, D_IN), jnp.float32),
        pltpu.SemaphoreType.DMA,
        pltpu.SemaphoreType.DMA,
    ],
)
def _sc_agg_h2(xlo, xhi, srcc, dstc, zacc,
               aggs, idxs, idxd, rows, acc, gsem, ssem):
    c = lax.axis_index("c")
    s = lax.axis_index("s")
    nck = NCHUNK // 16
    r0 = s * ZROWS
    pltpu.sync_copy(zacc.at[pl.ds(r0, ZROWS)], acc.at[pl.ds(r0, ZROWS)])
    plsc.subcore_barrier()

    def gstart(j, b):
        @pl.when(c == 0)
        def _():
            pltpu.async_copy(xlo.at[idxs.at[j]], rows.at[b], gsem)

        @pl.when(c == 1)
        def _():
            pltpu.async_copy(xhi.at[idxs.at[j]], rows.at[b], gsem)

    def gbody(g, carry):
        _edge_sweep(xlo, _GRP, gstart, srcc, dstc, s * nck + g * _GRP,
                    idxs, idxd, rows, acc, gsem, ssem)
        return carry

    lax.fori_loop(0, nck // _GRP, gbody, 0)
    plsc.subcore_barrier()

    @pl.when(c == 0)
    def _():
        pltpu.sync_copy(acc.at[pl.ds(r0, ZROWS)], aggs.at[0, pl.ds(r0, ZROWS)])

    @pl.when(c == 1)
    def _():
        pltpu.sync_copy(acc.at[pl.ds(r0, ZROWS)], aggs.at[1, pl.ds(r0, ZROWS)])


_BLK = 400
_EBLK = 8192
_NQ = 80


def _tc_deg_body(dstb, o):
    i = pl.program_id(0)
    dv = dstb[0]
    q = dv // 128
    r = dv % 128
    m1 = (lax.broadcasted_iota(jnp.int32, (_NQ, _EBLK), 0) == q
          ).astype(jnp.bfloat16)
    m2 = (lax.broadcasted_iota(jnp.int32, (128, _EBLK), 0) == r
          ).astype(jnp.bfloat16)
    part = lax.dot_general(m1, m2, (((1,), (1,)), ((), ())),
                           preferred_element_type=jnp.float32)

    @pl.when(i == 0)
    def _():
        o[...] = part

    @pl.when(i > 0)
    def _():
        o[...] = o[...] + part


_tc_deg = pl.pallas_call(
    _tc_deg_body,
    grid=(E_PAD // _EBLK,),
    in_specs=[pl.BlockSpec((1, 1, _EBLK), lambda i: (i, 0, 0))],
    out_specs=pl.BlockSpec((_NQ, 128), lambda i: (0, 0)),
    out_shape=jax.ShapeDtypeStruct((_NQ, 128), jnp.float32),
)


def _dotT(a, w):
    return lax.dot_general(a, w, (((1,), (1,)), ((), ())),
                           preferred_element_type=jnp.float32)


def _norm_act(h):
    n = jnp.sqrt(jnp.sum(h * h, axis=1, keepdims=True))
    h = h / jnp.maximum(n, 1e-12)
    return jnp.where(h > 0, h, 0.01 * h)


def _tc1_body(x0, aggp, degp, wl, wr, b, olo, ohi):
    deg = degp[0, 0]
    dinv = 1.0 / jnp.maximum(deg, 1.0)
    agg = (aggp[0] + aggp[1]) * dinv[:, None]
    h = _dotT(agg, wl[...]) + _dotT(x0[...], wr[...]) + b[...]
    h = _norm_act(h)
    olo[...] = h[:, :D_IN]
    ohi[...] = h[:, D_IN:]


_tc1 = pl.pallas_call(
    _tc1_body,
    grid=(N_NODES // _BLK,),
    in_specs=[
        pl.BlockSpec((_BLK, D_IN), lambda i: (i, 0)),
        pl.BlockSpec((2, _BLK, D_IN), lambda i: (0, i, 0)),
        pl.BlockSpec((1, 1, _BLK), lambda i: (i, 0, 0)),
        pl.BlockSpec((D_HID, D_IN), lambda i: (0, 0)),
        pl.BlockSpec((D_HID, D_IN), lambda i: (0, 0)),
        pl.BlockSpec((1, D_HID), lambda i: (0, 0)),
    ],
    out_specs=[pl.BlockSpec((_BLK, D_IN), lambda i: (i, 0)),
               pl.BlockSpec((_BLK, D_IN), lambda i: (i, 0))],
    out_shape=[jax.ShapeDtypeStruct((N_NODES, D_IN), jnp.float32),
               jax.ShapeDtypeStruct((N_NODES, D_IN), jnp.float32)],
)


def _make_tc23(final):
    def body(xlo, xhi, aggs, degp, wl, wr, b, *outs):
        deg = degp[0, 0]
        dinv = 1.0 / jnp.maximum(deg, 1.0)
        agg = jnp.concatenate([aggs[0], aggs[1]], axis=1) * dinv[:, None]
        x = jnp.concatenate([xlo[...], xhi[...]], axis=1)
        h = _dotT(agg, wl[...]) + _dotT(x, wr[...]) + b[...]
        if final:
            outs[0][...] = h
        else:
            h = _norm_act(h)
            outs[0][...] = h[:, :D_IN]
            outs[1][...] = h[:, D_IN:]

    in_specs = [
        pl.BlockSpec((_BLK, D_IN), lambda i: (i, 0)),
        pl.BlockSpec((_BLK, D_IN), lambda i: (i, 0)),
        pl.BlockSpec((2, _BLK, D_IN), lambda i: (0, i, 0)),
        pl.BlockSpec((1, 1, _BLK), lambda i: (i, 0, 0)),
        pl.BlockSpec((D_HID, D_HID), lambda i: (0, 0)),
        pl.BlockSpec((D_HID, D_HID), lambda i: (0, 0)),
        pl.BlockSpec((1, D_HID), lambda i: (0, 0)),
    ]
    if final:
        out_specs = pl.BlockSpec((_BLK, D_HID), lambda i: (i, 0))
        out_shape = jax.ShapeDtypeStruct((N_NODES, D_HID), jnp.float32)
    else:
        out_specs = [pl.BlockSpec((_BLK, D_IN), lambda i: (i, 0)),
                     pl.BlockSpec((_BLK, D_IN), lambda i: (i, 0))]
        out_shape = [jax.ShapeDtypeStruct((N_NODES, D_IN), jnp.float32),
                     jax.ShapeDtypeStruct((N_NODES, D_IN), jnp.float32)]
    return pl.pallas_call(body, grid=(N_NODES // _BLK,), in_specs=in_specs,
                          out_specs=out_specs, out_shape=out_shape)


_tc2 = _make_tc23(final=False)
_tc3 = _make_tc23(final=True)


def kernel(node_features, node_separation, node_ops, edges, batches,
           opcode_emb, W1l, b1, W1r, W2l, b2, W2r, W3l, b3, W3r):
    del node_separation, batches

    src = edges[0].astype(jnp.int32)
    dst = edges[1].astype(jnp.int32)
    pad_e = E_PAD - N_EDGES

    def _chunked(v, fill):
        c = jnp.concatenate(
            [v, jnp.full((pad_e,), fill, jnp.int32)]).reshape(NCHUNK, CHUNK)
        return c.reshape(NCHUNK // 32, 32, CHUNK).transpose(1, 0, 2).reshape(
            NCHUNK, CHUNK)

    srcc = _chunked(src, 0)
    dstc = _chunked(dst, PAD_DST)
    ops_p = jnp.concatenate(
        [node_ops.astype(jnp.int32),
         jnp.zeros((NOPS_PAD - N_NODES,), jnp.int32)])
    zacc = jnp.zeros((NACC, D_IN), jnp.float32)

    emb_p = jnp.concatenate(
        [opcode_emb, jnp.zeros((NUM_OPS, 128 - OP_DIM), jnp.float32)], axis=1)
    op_e = _sc_op_lookup(emb_p, ops_p)[:N_NODES, :OP_DIM]
    x0 = jnp.concatenate([node_features, op_e], axis=1)

    degq = _tc_deg(dstc.reshape(E_PAD // _EBLK, 1, _EBLK))
    degt = degq.reshape(_NQ * 128)[:N_NODES].reshape(N_NODES // _BLK, 1, _BLK)

    aggp = _sc_agg_l1(x0, srcc, dstc, zacc)
    x1lo, x1hi = _tc1(x0, aggp, degt, W1l, W1r, b1.reshape(1, D_HID))

    aggs2 = _sc_agg_h2(x1lo, x1hi, srcc, dstc, zacc)
    x2lo, x2hi = _tc2(x1lo, x1hi, aggs2, degt, W2l, W2r, b2.reshape(1, D_HID))

    aggs3 = _sc_agg_h2(x2lo, x2hi, srcc, dstc, zacc)
    return _tc3(x2lo, x2hi, aggs3, degt, W3l, W3r, b3.reshape(1, D_HID))


# speedup vs baseline: 2.2779x; 1.0195x over previous
import functools

import jax
import jax.numpy as jnp
from jax import lax
from jax.experimental import pallas as pl
from jax.experimental.pallas import tpu as pltpu
from jax.experimental.pallas import tpu_sc as plsc

N_NODES = 10000
N_EDGES = 320000
NUM_OPS = 120
OP_DIM = 64
D_IN = 128
D_HID = 256

CHUNK = 128
NCHUNK = 2560
E_PAD = NCHUNK * CHUNK
PAD_DST = 10104
NACC = 10112
ZROWS = NACC // 16
OPS_PER_TILE = 320
NOPS_PAD = 10240

_mesh = lambda: plsc.VectorSubcoreMesh(core_axis_name="c", subcore_axis_name="s")


@functools.partial(
    pl.kernel, mesh=_mesh(),
    out_type=jax.ShapeDtypeStruct((NOPS_PAD, 128), jnp.float32),
    scratch_types=[
        pltpu.VMEM((OPS_PER_TILE,), jnp.int32),
        pltpu.VMEM((CHUNK, 128), jnp.float32),
        pltpu.SemaphoreType.DMA,
    ],
)
def _sc_op_lookup(emb, ops, out, idxv, buf, sem):
    c = lax.axis_index("c")
    s = lax.axis_index("s")
    base = (c * 16 + s) * OPS_PER_TILE
    pltpu.sync_copy(ops.at[pl.ds(base, OPS_PER_TILE)], idxv)
    for off, sz in ((0, 128), (128, 128), (256, 64)):
        pltpu.async_copy(emb.at[idxv.at[pl.ds(off, sz)]],
                         buf.at[pl.ds(0, sz)], sem).wait()
        pltpu.sync_copy(buf.at[pl.ds(0, sz)], out.at[pl.ds(base + off, sz)])


def _edge_sweep(gtable, ngrp, grp, gstart, srcc, dstc, base0,
                idxs, idxd, rows, acc, gsem, ssem, isem):
    pltpu.sync_copy(srcc.at[pl.ds(base0, grp)], idxs.at[0])
    pltpu.sync_copy(dstc.at[pl.ds(base0, grp)], idxd.at[0])

    def gbody(g, carry):
        sl = lax.rem(g, 2)
        nsl = lax.rem(g + 1, 2)

        @pl.when(g + 1 < ngrp)
        def _():
            nb0 = base0 + (g + 1) * grp
            pltpu.async_copy(srcc.at[pl.ds(nb0, grp)], idxs.at[nsl], isem)
            pltpu.async_copy(dstc.at[pl.ds(nb0, grp)], idxd.at[nsl], isem)

        def gwait(j, b):
            pltpu.make_async_copy(
                gtable.at[idxs.at[sl, j]], rows.at[b], gsem).wait()

        gstart(sl, 0, 0)

        def body(j, carry2):
            b = lax.rem(j, 2)
            nb = lax.rem(j + 1, 2)

            @pl.when(j > 0)
            def _():
                pltpu.make_async_copy(
                    rows.at[nb], acc.at[idxd.at[sl, j - 1]], ssem).wait()

            @pl.when(j + 1 < grp)
            def _():
                gstart(sl, j + 1, nb)

            gwait(j, b)
            pltpu.async_copy(rows.at[b], acc.at[idxd.at[sl, j]], ssem,
                             add=True)
            return carry2

        lax.fori_loop(0, grp, body, 0)
        pltpu.make_async_copy(
            rows.at[(grp - 1) % 2], acc.at[idxd.at[sl, grp - 1]], ssem).wait()

        @pl.when(g + 1 < ngrp)
        def _():
            pltpu.make_async_copy(
                srcc.at[pl.ds(base0, grp)], idxs.at[nsl], isem).wait()
            pltpu.make_async_copy(
                dstc.at[pl.ds(base0, grp)], idxd.at[nsl], isem).wait()
        return carry

    lax.fori_loop(0, ngrp, gbody, 0)


_GRP1 = 16
_GRP = 32


@functools.partial(
    pl.kernel, mesh=_mesh(),
    out_type=jax.ShapeDtypeStruct((2, NACC, D_IN), jnp.float32),
    scratch_types=[
        pltpu.VMEM((2, _GRP1, CHUNK), jnp.int32),
        pltpu.VMEM((2, _GRP1, CHUNK), jnp.int32),
        pltpu.VMEM((2, CHUNK, D_IN), jnp.float32),
        pltpu.VMEM_SHARED((NACC, D_IN), jnp.float32),
        pltpu.SemaphoreType.DMA,
        pltpu.SemaphoreType.DMA,
        pltpu.SemaphoreType.DMA,
    ],
)
def _sc_agg_l1(x0, srcc, dstc, zacc,
               aggp, idxs, idxd, rows, acc, gsem, ssem, isem):
    c = lax.axis_index("c")
    s = lax.axis_index("s")
    t = c * 16 + s
    nck = NCHUNK // 32
    r0 = s * ZROWS
    pltpu.sync_copy(zacc.at[pl.ds(r0, ZROWS)], acc.at[pl.ds(r0, ZROWS)])
    plsc.subcore_barrier()

    def gstart(sl, j, b):
        pltpu.async_copy(x0.at[idxs.at[sl, j]], rows.at[b], gsem)

    _edge_sweep(x0, nck // _GRP1, _GRP1, gstart, srcc, dstc, t * nck,
                idxs, idxd, rows, acc, gsem, ssem, isem)
    plsc.subcore_barrier()

    @pl.when(c == 0)
    def _():
        pltpu.sync_copy(acc.at[pl.ds(r0, ZROWS)], aggp.at[0, pl.ds(r0, ZROWS)])

    @pl.when(c == 1)
    def _():
        pltpu.sync_copy(acc.at[pl.ds(r0, ZROWS)], aggp.at[1, pl.ds(r0, ZROWS)])


@functools.partial(
    pl.kernel, mesh=_mesh(),
    out_type=jax.ShapeDtypeStruct((2, NACC, D_IN), jnp.float32),
    scratch_types=[
        pltpu.VMEM((2, _GRP, CHUNK), jnp.int32),
        pltpu.VMEM((2, _GRP, CHUNK), jnp.int32),
        pltpu.VMEM((2, CHUNK, D_IN), jnp.float32),
        pltpu.VMEM_SHARED((NACC, D_IN), jnp.float32),
        pltpu.SemaphoreType.DMA,
        pltpu.SemaphoreType.DMA,
        pltpu.SemaphoreType.DMA,
    ],
)
def _sc_agg_h2(xlo, xhi, srcc, dstc, zacc,
               aggs, idxs, idxd, rows, acc, gsem, ssem, isem):
    c = lax.axis_index("c")
    s = lax.axis_index("s")
    nck = NCHUNK // 16
    r0 = s * ZROWS
    pltpu.sync_copy(zacc.at[pl.ds(r0, ZROWS)], acc.at[pl.ds(r0, ZROWS)])
    plsc.subcore_barrier()

    def gstart(sl, j, b):
        @pl.when(c == 0)
        def _():
            pltpu.async_copy(xlo.at[idxs.at[sl, j]], rows.at[b], gsem)

        @pl.when(c == 1)
        def _():
            pltpu.async_copy(xhi.at[idxs.at[sl, j]], rows.at[b], gsem)

    _edge_sweep(xlo, nck // _GRP, _GRP, gstart, srcc, dstc, s * nck,
                idxs, idxd, rows, acc, gsem, ssem, isem)
    plsc.subcore_barrier()

    @pl.when(c == 0)
    def _():
        pltpu.sync_copy(acc.at[pl.ds(r0, ZROWS)], aggs.at[0, pl.ds(r0, ZROWS)])

    @pl.when(c == 1)
    def _():
        pltpu.sync_copy(acc.at[pl.ds(r0, ZROWS)], aggs.at[1, pl.ds(r0, ZROWS)])


_BLK = 1000
_EBLK = 8192
_NQ = 80


def _tc_deg_body(dstb, o):
    i = pl.program_id(0)
    dv = dstb[0]
    q = dv // 128
    r = dv % 128
    m1 = (lax.broadcasted_iota(jnp.int32, (_NQ, _EBLK), 0) == q
          ).astype(jnp.bfloat16)
    m2 = (lax.broadcasted_iota(jnp.int32, (128, _EBLK), 0) == r
          ).astype(jnp.bfloat16)
    part = lax.dot_general(m1, m2, (((1,), (1,)), ((), ())),
                           preferred_element_type=jnp.float32)

    @pl.when(i == 0)
    def _():
        o[...] = part

    @pl.when(i > 0)
    def _():
        o[...] = o[...] + part


_tc_deg = pl.pallas_call(
    _tc_deg_body,
    grid=(E_PAD // _EBLK,),
    in_specs=[pl.BlockSpec((1, 1, _EBLK), lambda i: (i, 0, 0))],
    out_specs=pl.BlockSpec((_NQ, 128), lambda i: (0, 0)),
    out_shape=jax.ShapeDtypeStruct((_NQ, 128), jnp.float32),
)


def _dotT(a, w):
    return lax.dot_general(a, w, (((1,), (1,)), ((), ())),
                           preferred_element_type=jnp.float32)


def _norm_act(h):
    n = jnp.sqrt(jnp.sum(h * h, axis=1, keepdims=True))
    h = h / jnp.maximum(n, 1e-12)
    return jnp.where(h > 0, h, 0.01 * h)


def _tc1_body(x0, aggp, degp, wl, wr, b, olo, ohi):
    deg = degp[0, 0]
    dinv = 1.0 / jnp.maximum(deg, 1.0)
    agg = (aggp[0] + aggp[1]) * dinv[:, None]
    h = _dotT(agg, wl[...]) + _dotT(x0[...], wr[...]) + b[...]
    h = _norm_act(h)
    olo[...] = h[:, :D_IN]
    ohi[...] = h[:, D_IN:]


_tc1 = pl.pallas_call(
    _tc1_body,
    grid=(N_NODES // _BLK,),
    in_specs=[
        pl.BlockSpec((_BLK, D_IN), lambda i: (i, 0)),
        pl.BlockSpec((2, _BLK, D_IN), lambda i: (0, i, 0)),
        pl.BlockSpec((1, 1, _BLK), lambda i: (i, 0, 0)),
        pl.BlockSpec((D_HID, D_IN), lambda i: (0, 0)),
        pl.BlockSpec((D_HID, D_IN), lambda i: (0, 0)),
        pl.BlockSpec((1, D_HID), lambda i: (0, 0)),
    ],
    out_specs=[pl.BlockSpec((_BLK, D_IN), lambda i: (i, 0)),
               pl.BlockSpec((_BLK, D_IN), lambda i: (i, 0))],
    out_shape=[jax.ShapeDtypeStruct((N_NODES, D_IN), jnp.float32),
               jax.ShapeDtypeStruct((N_NODES, D_IN), jnp.float32)],
)


def _make_tc23(final):
    def body(xlo, xhi, aggs, degp, wl, wr, b, *outs):
        deg = degp[0, 0]
        dinv = 1.0 / jnp.maximum(deg, 1.0)
        agg = jnp.concatenate([aggs[0], aggs[1]], axis=1) * dinv[:, None]
        x = jnp.concatenate([xlo[...], xhi[...]], axis=1)
        h = _dotT(agg, wl[...]) + _dotT(x, wr[...]) + b[...]
        if final:
            outs[0][...] = h
        else:
            h = _norm_act(h)
            outs[0][...] = h[:, :D_IN]
            outs[1][...] = h[:, D_IN:]

    in_specs = [
        pl.BlockSpec((_BLK, D_IN), lambda i: (i, 0)),
        pl.BlockSpec((_BLK, D_IN), lambda i: (i, 0)),
        pl.BlockSpec((2, _BLK, D_IN), lambda i: (0, i, 0)),
        pl.BlockSpec((1, 1, _BLK), lambda i: (i, 0, 0)),
        pl.BlockSpec((D_HID, D_HID), lambda i: (0, 0)),
        pl.BlockSpec((D_HID, D_HID), lambda i: (0, 0)),
        pl.BlockSpec((1, D_HID), lambda i: (0, 0)),
    ]
    if final:
        out_specs = pl.BlockSpec((_BLK, D_HID), lambda i: (i, 0))
        out_shape = jax.ShapeDtypeStruct((N_NODES, D_HID), jnp.float32)
    else:
        out_specs = [pl.BlockSpec((_BLK, D_IN), lambda i: (i, 0)),
                     pl.BlockSpec((_BLK, D_IN), lambda i: (i, 0))]
        out_shape = [jax.ShapeDtypeStruct((N_NODES, D_IN), jnp.float32),
                     jax.ShapeDtypeStruct((N_NODES, D_IN), jnp.float32)]
    return pl.pallas_call(body, grid=(N_NODES // _BLK,), in_specs=in_specs,
                          out_specs=out_specs, out_shape=out_shape)


_tc2 = _make_tc23(final=False)
_tc3 = _make_tc23(final=True)


def kernel(node_features, node_separation, node_ops, edges, batches,
           opcode_emb, W1l, b1, W1r, W2l, b2, W2r, W3l, b3, W3r):
    del node_separation, batches

    src = edges[0].astype(jnp.int32)
    dst = edges[1].astype(jnp.int32)
    pad_e = E_PAD - N_EDGES

    def _chunked(v, fill):
        c = jnp.concatenate(
            [v, jnp.full((pad_e,), fill, jnp.int32)]).reshape(NCHUNK, CHUNK)
        return c.reshape(NCHUNK // 32, 32, CHUNK).transpose(1, 0, 2).reshape(
            NCHUNK, CHUNK)

    srcc = _chunked(src, 0)
    dstc = _chunked(dst, PAD_DST)
    ops_p = jnp.concatenate(
        [node_ops.astype(jnp.int32),
         jnp.zeros((NOPS_PAD - N_NODES,), jnp.int32)])
    zacc = jnp.zeros((NACC, D_IN), jnp.float32)

    emb_p = jnp.concatenate(
        [opcode_emb, jnp.zeros((NUM_OPS, 128 - OP_DIM), jnp.float32)], axis=1)
    op_e = _sc_op_lookup(emb_p, ops_p)[:N_NODES, :OP_DIM]
    x0 = jnp.concatenate([node_features, op_e], axis=1)

    degq = _tc_deg(dstc.reshape(E_PAD // _EBLK, 1, _EBLK))
    degt = degq.reshape(_NQ * 128)[:N_NODES].reshape(N_NODES // _BLK, 1, _BLK)

    aggp = _sc_agg_l1(x0, srcc, dstc, zacc)
    x1lo, x1hi = _tc1(x0, aggp, degt, W1l, W1r, b1.reshape(1, D_HID))

    aggs2 = _sc_agg_h2(x1lo, x1hi, srcc, dstc, zacc)
    x2lo, x2hi = _tc2(x1lo, x1hi, aggs2, degt, W2l, W2r, b2.reshape(1, D_HID))

    aggs3 = _sc_agg_h2(x2lo, x2hi, srcc, dstc, zacc)
    return _tc3(x2lo, x2hi, aggs3, degt, W3l, W3r, b3.reshape(1, D_HID))


# speedup vs baseline: 2.2786x; 1.0003x over previous
import functools

import jax
import jax.numpy as jnp
from jax import lax
from jax.experimental import pallas as pl
from jax.experimental.pallas import tpu as pltpu
from jax.experimental.pallas import tpu_sc as plsc

N_NODES = 10000
N_EDGES = 320000
NUM_OPS = 120
OP_DIM = 64
D_IN = 128
D_HID = 256

CHUNK = 128
NCHUNK = 2560
E_PAD = NCHUNK * CHUNK
PAD_DST = 10104
NACC = 10112
ZROWS = NACC // 16
OPS_PER_TILE = 320
NOPS_PAD = 10240

_mesh = lambda: plsc.VectorSubcoreMesh(core_axis_name="c", subcore_axis_name="s")


@functools.partial(
    pl.kernel, mesh=_mesh(),
    out_type=jax.ShapeDtypeStruct((NOPS_PAD, 128), jnp.float32),
    scratch_types=[
        pltpu.VMEM((OPS_PER_TILE,), jnp.int32),
        pltpu.VMEM((CHUNK, 128), jnp.float32),
        pltpu.SemaphoreType.DMA,
    ],
)
def _sc_op_lookup(emb, ops, out, idxv, buf, sem):
    c = lax.axis_index("c")
    s = lax.axis_index("s")
    base = (c * 16 + s) * OPS_PER_TILE
    pltpu.sync_copy(ops.at[pl.ds(base, OPS_PER_TILE)], idxv)
    for off, sz in ((0, 128), (128, 128), (256, 64)):
        pltpu.async_copy(emb.at[idxv.at[pl.ds(off, sz)]],
                         buf.at[pl.ds(0, sz)], sem).wait()
        pltpu.sync_copy(buf.at[pl.ds(0, sz)], out.at[pl.ds(base + off, sz)])


def _edge_sweep(gtable, ngrp, grp, gstart, srcc, dstc, base0,
                idxs, idxd, rows, acc, gsem, ssem, isem):
    pltpu.sync_copy(srcc.at[pl.ds(base0, grp)], idxs.at[0])
    pltpu.sync_copy(dstc.at[pl.ds(base0, grp)], idxd.at[0])

    def gbody(g, carry):
        sl = lax.rem(g, 2)
        nsl = lax.rem(g + 1, 2)

        @pl.when(g + 1 < ngrp)
        def _():
            nb0 = base0 + (g + 1) * grp
            pltpu.async_copy(srcc.at[pl.ds(nb0, grp)], idxs.at[nsl], isem)
            pltpu.async_copy(dstc.at[pl.ds(nb0, grp)], idxd.at[nsl], isem)

        def gwait(j, b):
            pltpu.make_async_copy(
                gtable.at[idxs.at[sl, j]], rows.at[b], gsem).wait()

        gstart(sl, 0, 0)

        def body(j, carry2):
            b = lax.rem(j, 2)
            nb = lax.rem(j + 1, 2)

            @pl.when(j > 0)
            def _():
                pltpu.make_async_copy(
                    rows.at[nb], acc.at[idxd.at[sl, j - 1]], ssem).wait()

            @pl.when(j + 1 < grp)
            def _():
                gstart(sl, j + 1, nb)

            gwait(j, b)
            pltpu.async_copy(rows.at[b], acc.at[idxd.at[sl, j]], ssem,
                             add=True)
            return carry2

        lax.fori_loop(0, grp, body, 0)
        pltpu.make_async_copy(
            rows.at[(grp - 1) % 2], acc.at[idxd.at[sl, grp - 1]], ssem).wait()

        @pl.when(g + 1 < ngrp)
        def _():
            pltpu.make_async_copy(
                srcc.at[pl.ds(base0, grp)], idxs.at[nsl], isem).wait()
            pltpu.make_async_copy(
                dstc.at[pl.ds(base0, grp)], idxd.at[nsl], isem).wait()
        return carry

    lax.fori_loop(0, ngrp, gbody, 0)


_GRP1 = 16
_GRP = 32


@functools.partial(
    pl.kernel, mesh=_mesh(),
    out_type=jax.ShapeDtypeStruct((2, NACC, D_IN), jnp.float32),
    scratch_types=[
        pltpu.VMEM((2, _GRP1, CHUNK), jnp.int32),
        pltpu.VMEM((2, _GRP1, CHUNK), jnp.int32),
        pltpu.VMEM((2, CHUNK, D_IN), jnp.float32),
        pltpu.VMEM_SHARED((NACC, D_IN), jnp.float32),
        pltpu.SemaphoreType.DMA,
        pltpu.SemaphoreType.DMA,
        pltpu.SemaphoreType.DMA,
    ],
)
def _sc_agg_l1(x0a, x0b, srcc, dstc, zacc,
               aggp, idxs, idxd, rows, acc, gsem, ssem, isem):
    c = lax.axis_index("c")
    s = lax.axis_index("s")
    t = c * 16 + s
    nck = NCHUNK // 32
    r0 = s * ZROWS
    pltpu.sync_copy(zacc.at[pl.ds(r0, ZROWS)], acc.at[pl.ds(r0, ZROWS)])
    plsc.subcore_barrier()

    def gstart(sl, j, b):
        @pl.when(c == 0)
        def _():
            pltpu.async_copy(x0a.at[idxs.at[sl, j]], rows.at[b], gsem)

        @pl.when(c == 1)
        def _():
            pltpu.async_copy(x0b.at[idxs.at[sl, j]], rows.at[b], gsem)

    _edge_sweep(x0a, nck // _GRP1, _GRP1, gstart, srcc, dstc, t * nck,
                idxs, idxd, rows, acc, gsem, ssem, isem)
    plsc.subcore_barrier()

    @pl.when(c == 0)
    def _():
        pltpu.sync_copy(acc.at[pl.ds(r0, ZROWS)], aggp.at[0, pl.ds(r0, ZROWS)])

    @pl.when(c == 1)
    def _():
        pltpu.sync_copy(acc.at[pl.ds(r0, ZROWS)], aggp.at[1, pl.ds(r0, ZROWS)])


@functools.partial(
    pl.kernel, mesh=_mesh(),
    out_type=jax.ShapeDtypeStruct((2, NACC, D_IN), jnp.float32),
    scratch_types=[
        pltpu.VMEM((2, _GRP, CHUNK), jnp.int32),
        pltpu.VMEM((2, _GRP, CHUNK), jnp.int32),
        pltpu.VMEM((2, CHUNK, D_IN), jnp.float32),
        pltpu.VMEM_SHARED((NACC, D_IN), jnp.float32),
        pltpu.SemaphoreType.DMA,
        pltpu.SemaphoreType.DMA,
        pltpu.SemaphoreType.DMA,
    ],
)
def _sc_agg_h2(xlo, xhi, srcc, dstc, zacc,
               aggs, idxs, idxd, rows, acc, gsem, ssem, isem):
    c = lax.axis_index("c")
    s = lax.axis_index("s")
    nck = NCHUNK // 16
    r0 = s * ZROWS
    pltpu.sync_copy(zacc.at[pl.ds(r0, ZROWS)], acc.at[pl.ds(r0, ZROWS)])
    plsc.subcore_barrier()

    def gstart(sl, j, b):
        @pl.when(c == 0)
        def _():
            pltpu.async_copy(xlo.at[idxs.at[sl, j]], rows.at[b], gsem)

        @pl.when(c == 1)
        def _():
            pltpu.async_copy(xhi.at[idxs.at[sl, j]], rows.at[b], gsem)

    _edge_sweep(xlo, nck // _GRP, _GRP, gstart, srcc, dstc, s * nck,
                idxs, idxd, rows, acc, gsem, ssem, isem)
    plsc.subcore_barrier()

    @pl.when(c == 0)
    def _():
        pltpu.sync_copy(acc.at[pl.ds(r0, ZROWS)], aggs.at[0, pl.ds(r0, ZROWS)])

    @pl.when(c == 1)
    def _():
        pltpu.sync_copy(acc.at[pl.ds(r0, ZROWS)], aggs.at[1, pl.ds(r0, ZROWS)])


_BLK = 1000
_EBLK = 8192
_NQ = 80


def _tc_deg_body(dstb, o):
    i = pl.program_id(0)
    dv = dstb[0]
    q = dv // 128
    r = dv % 128
    m1 = (lax.broadcasted_iota(jnp.int32, (_NQ, _EBLK), 0) == q
          ).astype(jnp.bfloat16)
    m2 = (lax.broadcasted_iota(jnp.int32, (128, _EBLK), 0) == r
          ).astype(jnp.bfloat16)
    part = lax.dot_general(m1, m2, (((1,), (1,)), ((), ())),
                           preferred_element_type=jnp.float32)

    @pl.when(i == 0)
    def _():
        o[...] = part

    @pl.when(i > 0)
    def _():
        o[...] = o[...] + part


_tc_deg = pl.pallas_call(
    _tc_deg_body,
    grid=(E_PAD // _EBLK,),
    in_specs=[pl.BlockSpec((1, 1, _EBLK), lambda i: (i, 0, 0))],
    out_specs=pl.BlockSpec((_NQ, 128), lambda i: (0, 0)),
    out_shape=jax.ShapeDtypeStruct((_NQ, 128), jnp.float32),
)


def _dotT(a, w):
    return lax.dot_general(a, w, (((1,), (1,)), ((), ())),
                           preferred_element_type=jnp.float32)


def _norm_act(h):
    n = jnp.sqrt(jnp.sum(h * h, axis=1, keepdims=True))
    h = h / jnp.maximum(n, 1e-12)
    return jnp.where(h > 0, h, 0.01 * h)


def _tc1_body(x0, aggp, degp, wl, wr, b, olo, ohi):
    deg = degp[0, 0]
    dinv = 1.0 / jnp.maximum(deg, 1.0)
    agg = (aggp[0] + aggp[1]) * dinv[:, None]
    h = _dotT(agg, wl[...]) + _dotT(x0[...], wr[...]) + b[...]
    h = _norm_act(h)
    olo[...] = h[:, :D_IN]
    ohi[...] = h[:, D_IN:]


_tc1 = pl.pallas_call(
    _tc1_body,
    grid=(N_NODES // _BLK,),
    in_specs=[
        pl.BlockSpec((_BLK, D_IN), lambda i: (i, 0)),
        pl.BlockSpec((2, _BLK, D_IN), lambda i: (0, i, 0)),
        pl.BlockSpec((1, 1, _BLK), lambda i: (i, 0, 0)),
        pl.BlockSpec((D_HID, D_IN), lambda i: (0, 0)),
        pl.BlockSpec((D_HID, D_IN), lambda i: (0, 0)),
        pl.BlockSpec((1, D_HID), lambda i: (0, 0)),
    ],
    out_specs=[pl.BlockSpec((_BLK, D_IN), lambda i: (i, 0)),
               pl.BlockSpec((_BLK, D_IN), lambda i: (i, 0))],
    out_shape=[jax.ShapeDtypeStruct((N_NODES, D_IN), jnp.float32),
               jax.ShapeDtypeStruct((N_NODES, D_IN), jnp.float32)],
)


def _make_tc23(final):
    def body(xlo, xhi, aggs, degp, wl, wr, b, *outs):
        deg = degp[0, 0]
        dinv = 1.0 / jnp.maximum(deg, 1.0)
        agg = jnp.concatenate([aggs[0], aggs[1]], axis=1) * dinv[:, None]
        x = jnp.concatenate([xlo[...], xhi[...]], axis=1)
        h = _dotT(agg, wl[...]) + _dotT(x, wr[...]) + b[...]
        if final:
            outs[0][...] = h
        else:
            h = _norm_act(h)
            outs[0][...] = h[:, :D_IN]
            outs[1][...] = h[:, D_IN:]

    in_specs = [
        pl.BlockSpec((_BLK, D_IN), lambda i: (i, 0)),
        pl.BlockSpec((_BLK, D_IN), lambda i: (i, 0)),
        pl.BlockSpec((2, _BLK, D_IN), lambda i: (0, i, 0)),
        pl.BlockSpec((1, 1, _BLK), lambda i: (i, 0, 0)),
        pl.BlockSpec((D_HID, D_HID), lambda i: (0, 0)),
        pl.BlockSpec((D_HID, D_HID), lambda i: (0, 0)),
        pl.BlockSpec((1, D_HID), lambda i: (0, 0)),
    ]
    if final:
        out_specs = pl.BlockSpec((_BLK, D_HID), lambda i: (i, 0))
        out_shape = jax.ShapeDtypeStruct((N_NODES, D_HID), jnp.float32)
    else:
        out_specs = [pl.BlockSpec((_BLK, D_IN), lambda i: (i, 0)),
                     pl.BlockSpec((_BLK, D_IN), lambda i: (i, 0))]
        out_shape = [jax.ShapeDtypeStruct((N_NODES, D_IN), jnp.float32),
                     jax.ShapeDtypeStruct((N_NODES, D_IN), jnp.float32)]
    return pl.pallas_call(body, grid=(N_NODES // _BLK,), in_specs=in_specs,
                          out_specs=out_specs, out_shape=out_shape)


_tc2 = _make_tc23(final=False)
_tc3 = _make_tc23(final=True)


def kernel(node_features, node_separation, node_ops, edges, batches,
           opcode_emb, W1l, b1, W1r, W2l, b2, W2r, W3l, b3, W3r):
    del node_separation, batches

    src = edges[0].astype(jnp.int32)
    dst = edges[1].astype(jnp.int32)
    pad_e = E_PAD - N_EDGES

    def _chunked(v, fill):
        c = jnp.concatenate(
            [v, jnp.full((pad_e,), fill, jnp.int32)]).reshape(NCHUNK, CHUNK)
        return c.reshape(NCHUNK // 32, 32, CHUNK).transpose(1, 0, 2).reshape(
            NCHUNK, CHUNK)

    srcc = _chunked(src, 0)
    dstc = _chunked(dst, PAD_DST)
    ops_p = jnp.concatenate(
        [node_ops.astype(jnp.int32),
         jnp.zeros((NOPS_PAD - N_NODES,), jnp.int32)])
    zacc = jnp.zeros((NACC, D_IN), jnp.float32)

    emb_p = jnp.concatenate(
        [opcode_emb, jnp.zeros((NUM_OPS, 128 - OP_DIM), jnp.float32)], axis=1)
    op_e = _sc_op_lookup(emb_p, ops_p)[:N_NODES, :OP_DIM]
    x0 = jnp.concatenate([node_features, op_e], axis=1)

    degq = _tc_deg(dstc.reshape(E_PAD // _EBLK, 1, _EBLK))
    degt = degq.reshape(_NQ * 128)[:N_NODES].reshape(N_NODES // _BLK, 1, _BLK)

    x0b = jnp.copy(x0)
    aggp = _sc_agg_l1(x0, x0b, srcc, dstc, zacc)
    x1lo, x1hi = _tc1(x0, aggp, degt, W1l, W1r, b1.reshape(1, D_HID))

    aggs2 = _sc_agg_h2(x1lo, x1hi, srcc, dstc, zacc)
    x2lo, x2hi = _tc2(x1lo, x1hi, aggs2, degt, W2l, W2r, b2.reshape(1, D_HID))

    aggs3 = _sc_agg_h2(x2lo, x2hi, srcc, dstc, zacc)
    return _tc3(x2lo, x2hi, aggs3, degt, W3l, W3r, b3.reshape(1, D_HID))
